# baseline (speedup 1.0000x reference)
import jax
import jax.numpy as jnp
from jax import lax
from jax.experimental import pallas as pl
from jax.experimental.pallas import tpu as pltpu
from jax.experimental.pallas import tpu_sc as plsc

H, W = 256, 256
N_PIXELS = H * W
N_ENTRIES = 1000000
N_GENES = 20000
D = 64

NC, NS, L = 2, 16, 16
NW = NC * NS
PX_PER_W = N_PIXELS // NW
NPX = 1024
SB = PX_PER_W // NPX
B = 128
NGROUP = B // L
BSTEPS = 10
ZR = 64
R = 4


def _body(icnt_hbm, iptr_hbm, gamma_hbm, tok_hbm, out_hbm,
          gamma_v, iptr_v, end_v, icnt_v, c_v, pix_v, rows_v, zero_v,
          acc_sh,
          sem_i0, sem_i1, sem_i2, sem_i3, sem_c0, sem_c1, sem_c2, sem_c3,
          sem_g0, sem_g1, sem_g2, sem_g3, sem_s0, sem_s1, sem_s2, sem_s3):
  cid = lax.axis_index("c")
  sid = lax.axis_index("s")
  wid = cid * NS + sid
  sem_i = (sem_i0, sem_i1, sem_i2, sem_i3)
  sem_c = (sem_c0, sem_c1, sem_c2, sem_c3)
  sem_g = (sem_g0, sem_g1, sem_g2, sem_g3)
  sem_s = (sem_s0, sem_s1, sem_s2, sem_s3)

  pltpu.sync_copy(gamma_hbm, gamma_v)

  def _zrow(i, _):
    for j in range(D // L):
      zero_v[i, pl.ds(j * L, L)] = jnp.zeros((L,), jnp.float32)
    return 0
  lax.fori_loop(0, ZR, _zrow, 0)

  acc_base = sid * NPX

  def _sub_block(sb, _):
    p0 = wid * PX_PER_W + sb * NPX
    pltpu.sync_copy(iptr_hbm.at[pl.ds(p0, NPX)], iptr_v)
    pltpu.sync_copy(iptr_hbm.at[pl.ds(p0 + NPX, L)], end_v)
    start = iptr_v[pl.ds(0, L)][0]
    end = end_v[...][0]

    for q in range(NPX // ZR):
      pltpu.async_copy(zero_v, acc_sh.at[pl.ds(acc_base + q * ZR, ZR)],
                       sem_i0)
    for q in range(NPX // ZR):
      pltpu.make_async_copy(
          zero_v, acc_sh.at[pl.ds(acc_base + q * ZR, ZR)], sem_i0).wait()

    e0 = (start // 8) * 8
    n = (end - e0 + (B - 1)) // B

    def _idx_start(j, p):
      eb = e0 + j * B
      pltpu.async_copy(icnt_hbm.at[:, pl.ds(eb, B)], icnt_v.at[p], sem_i[p])

    def _idx_wait(j, p):
      eb = e0 + j * B
      pltpu.make_async_copy(
          icnt_hbm.at[:, pl.ds(eb, B)], icnt_v.at[p], sem_i[p]).wait()

    def _g_start(p):
      pltpu.async_copy(tok_hbm.at[icnt_v.at[p].at[0]], rows_v.at[p], sem_g[p])

    def _g_wait(p):
      pltpu.make_async_copy(
          tok_hbm.at[icnt_v.at[p].at[0]], rows_v.at[p], sem_g[p]).wait()

    def _s_start(p):
      pltpu.async_copy(rows_v.at[p], acc_sh.at[pix_v.at[p]], sem_s[p],
                       add=True)

    def _s_wait(p):
      pltpu.make_async_copy(
          rows_v.at[p], acc_sh.at[pix_v.at[p]], sem_s[p]).wait()

    def _comp(k, p):
      eb = e0 + k * B
      start_s = jnp.full((L,), start, jnp.int32)
      end_s = jnp.full((L,), end, jnp.int32)

      @plsc.parallel_loop(0, NGROUP, step=1, unroll=4)
      def _group(g):
        off = g * L
        idx16 = icnt_v[p, 0, pl.ds(off, L)]
        gam16 = plsc.load_gather(gamma_v, [idx16])
        e16 = eb + off + lax.iota(jnp.int32, L)
        cnt16 = plsc.bitcast(icnt_v[p, 1, pl.ds(off, L)], jnp.float32)
        c16 = cnt16 * jnp.exp(gam16)
        valid = (e16 >= start_s) & (e16 < end_s)
        c_v[p, pl.ds(off, L)] = jnp.where(
            valid, c16, jnp.zeros((L,), jnp.float32))
        lo = jnp.zeros((L,), jnp.int32)
        hi = jnp.full((L,), NPX, jnp.int32)
        for _t in range(BSTEPS):
          mid = (lo + hi) // 2
          le = plsc.load_gather(iptr_v, [mid]) <= e16
          lo = jnp.where(le, mid, lo)
          hi = jnp.where(le, hi, mid)
        pix_v[p, pl.ds(off, L)] = lo + acc_base

    def _scale(p):
      @plsc.parallel_loop(0, B, step=1, unroll=8)
      def _one(b):
        cb = plsc.load_gather(c_v.at[p], [jnp.full((L,), b, jnp.int32)])
        for j in range(D // L):
          sl = pl.ds(j * L, L)
          rows_v[p, b, sl] = rows_v[p, b, sl] * cb

    def _stage(k, j):
      pw = (j + 2) % R
      pn = (j + 3) % R

      @pl.when((k >= 2) & (k - 2 < n))
      def _():
        _s_wait(pw)

      @pl.when(k < n)
      def _():
        _comp(k, j)

        @pl.when(k + 2 < n)
        def _():
          _idx_wait(k + 2, pw)
          _g_start(pw)

        @pl.when(k + 3 < n)
        def _():
          _idx_start(k + 3, pn)

        _g_wait(j)
        _scale(j)
        _s_start(j)

    for j in range(3):
      @pl.when(j < n)
      def _(j=j):
        _idx_start(j, j)
    for j in range(2):
      @pl.when(j < n)
      def _(j=j):
        _idx_wait(j, j)
        _g_start(j)

    def _quad(m, _):
      k = R * m
      for j in range(R):
        _stage(k + j, j)
      return 0
    lax.fori_loop(0, (n + 2 + (R - 1)) // R, _quad, 0)

    pltpu.sync_copy(acc_sh.at[pl.ds(acc_base, NPX)], out_hbm.at[pl.ds(p0, NPX)])
    return 0
  lax.fori_loop(0, SB, _sub_block, 0)


@jax.jit
def kernel(indices, cnts, indptr, gamma, tokens):
  idx_p = jnp.concatenate([indices, jnp.zeros((3 * B,), jnp.int32)])
  cnt_p = jnp.concatenate([cnts, jnp.zeros((3 * B,), jnp.float32)])
  icnt = jnp.stack([idx_p, jax.lax.bitcast_convert_type(cnt_p, jnp.int32)])
  iptr_p = jnp.concatenate(
      [indptr, jnp.full((L - 1,), N_ENTRIES, jnp.int32)])

  mesh = plsc.VectorSubcoreMesh(
      core_axis_name="c", subcore_axis_name="s", num_cores=NC,
      num_subcores=NS)
  run = pl.kernel(
      _body,
      out_type=jax.ShapeDtypeStruct((N_PIXELS, D), jnp.float32),
      mesh=mesh,
      compiler_params=pltpu.CompilerParams(
          needs_layout_passes=False, use_tc_tiling_on_sc=False),
      scratch_types=[
          pltpu.VMEM((N_GENES,), jnp.float32),
          pltpu.VMEM((NPX,), jnp.int32),
          pltpu.VMEM((L,), jnp.int32),
          pltpu.VMEM((R, 2, B), jnp.int32),
          pltpu.VMEM((R, B), jnp.float32),
          pltpu.VMEM((R, B), jnp.int32),
          pltpu.VMEM((R, B, D), jnp.float32),
          pltpu.VMEM((ZR, D), jnp.float32),
          pltpu.VMEM_SHARED((NS * NPX, D), jnp.float32),
      ] + [pltpu.SemaphoreType.DMA] * 16,
  )
  out = run(icnt, iptr_p, gamma, tokens)
  return out.reshape(H, W, D)


# speedup vs baseline: 1.0766x; 1.0766x over previous
import jax
import jax.numpy as jnp
from jax import lax
from jax.experimental import pallas as pl
from jax.experimental.pallas import tpu as pltpu
from jax.experimental.pallas import tpu_sc as plsc

H, W = 256, 256
N_PIXELS = H * W
N_ENTRIES = 1000000
N_GENES = 20000
D = 64

NC, NS, L = 2, 16, 16
NW = NC * NS
PX_PER_W = N_PIXELS // NW
NPX = 1024
SB = PX_PER_W // NPX
B = 128
NGROUP = B // L
BSTEPS = 10
ZR = 64
R = 4


def _body(idx_hbm, cnt_hbm, iptr_hbm, gamma_hbm, tok_hbm, out_hbm,
          gamma_v, iptr_v, end_v, idx_v, cnt_v, c_v, pix_v, rows_v, zero_v,
          acc_sh,
          sem_i0, sem_i1, sem_i2, sem_i3, sem_c0, sem_c1, sem_c2, sem_c3,
          sem_g0, sem_g1, sem_g2, sem_g3, sem_s0, sem_s1, sem_s2, sem_s3):
  cid = lax.axis_index("c")
  sid = lax.axis_index("s")
  wid = cid * NS + sid
  sem_i = (sem_i0, sem_i1, sem_i2, sem_i3)
  sem_c = (sem_c0, sem_c1, sem_c2, sem_c3)
  sem_g = (sem_g0, sem_g1, sem_g2, sem_g3)
  sem_s = (sem_s0, sem_s1, sem_s2, sem_s3)

  pltpu.sync_copy(gamma_hbm, gamma_v)

  def _zrow(i, _):
    for j in range(D // L):
      zero_v[i, pl.ds(j * L, L)] = jnp.zeros((L,), jnp.float32)
    return 0
  lax.fori_loop(0, ZR, _zrow, 0)

  acc_base = sid * NPX

  def _sub_block(sb, _):
    p0 = wid * PX_PER_W + sb * NPX
    pltpu.sync_copy(iptr_hbm.at[pl.ds(p0, NPX)], iptr_v)
    pltpu.sync_copy(iptr_hbm.at[pl.ds(p0 + NPX, L)], end_v)
    start = iptr_v[pl.ds(0, L)][0]
    end = end_v[...][0]

    for q in range(NPX // ZR):
      pltpu.async_copy(zero_v, acc_sh.at[pl.ds(acc_base + q * ZR, ZR)],
                       sem_i0)
    for q in range(NPX // ZR):
      pltpu.make_async_copy(
          zero_v, acc_sh.at[pl.ds(acc_base + q * ZR, ZR)], sem_i0).wait()

    e0 = (start // 8) * 8
    n = (end - e0 + (B - 1)) // B

    def _idx_start(j, p):
      eb = e0 + j * B
      pltpu.async_copy(idx_hbm.at[pl.ds(eb, B)], idx_v.at[p], sem_i[p])
      pltpu.async_copy(cnt_hbm.at[pl.ds(eb, B)], cnt_v.at[p], sem_c[p])

    def _idx_wait(j, p):
      eb = e0 + j * B
      pltpu.make_async_copy(
          idx_hbm.at[pl.ds(eb, B)], idx_v.at[p], sem_i[p]).wait()
      pltpu.make_async_copy(
          cnt_hbm.at[pl.ds(eb, B)], cnt_v.at[p], sem_c[p]).wait()

    def _g_start(p):
      pltpu.async_copy(tok_hbm.at[idx_v.at[p]], rows_v.at[p], sem_g[p])

    def _g_wait(p):
      pltpu.make_async_copy(
          tok_hbm.at[idx_v.at[p]], rows_v.at[p], sem_g[p]).wait()

    def _s_start(p):
      pltpu.async_copy(rows_v.at[p], acc_sh.at[pix_v.at[p]], sem_s[p],
                       add=True)

    def _s_wait(p):
      pltpu.make_async_copy(
          rows_v.at[p], acc_sh.at[pix_v.at[p]], sem_s[p]).wait()

    def _comp_scale(k, p):
      eb = e0 + k * B
      start_s = jnp.full((L,), start, jnp.int32)
      end_s = jnp.full((L,), end, jnp.int32)

      @plsc.parallel_loop(0, NGROUP, step=1, unroll=2)
      def _group(g):
        off = g * L
        idx16 = idx_v[p, pl.ds(off, L)]
        gam16 = plsc.load_gather(gamma_v, [idx16])
        e16 = eb + off + lax.iota(jnp.int32, L)
        c16 = cnt_v[p, pl.ds(off, L)] * jnp.exp(gam16)
        valid = (e16 >= start_s) & (e16 < end_s)
        c16 = jnp.where(valid, c16, jnp.zeros((L,), jnp.float32))
        lo = jnp.zeros((L,), jnp.int32)
        hi = jnp.full((L,), NPX, jnp.int32)
        for _t in range(BSTEPS):
          mid = (lo + hi) // 2
          le = plsc.load_gather(iptr_v, [mid]) <= e16
          lo = jnp.where(le, mid, lo)
          hi = jnp.where(le, hi, mid)
        pix_v[p, pl.ds(off, L)] = lo + acc_base
        for b in range(L):
          cb = jnp.full((L,), c16[b], jnp.float32)
          for j in range(D // L):
            sl = pl.ds(j * L, L)
            rows_v[p, off + b, sl] = rows_v[p, off + b, sl] * cb

    def _stage(k, j):
      pw = (j + 2) % R
      pn = (j + 3) % R

      @pl.when((k >= 2) & (k - 2 < n))
      def _():
        _s_wait(pw)

      @pl.when(k < n)
      def _():
        @pl.when(k + 2 < n)
        def _():
          _idx_wait(k + 2, pw)
          _g_start(pw)

        @pl.when(k + 3 < n)
        def _():
          _idx_start(k + 3, pn)

        _g_wait(j)
        _comp_scale(k, j)
        _s_start(j)

    for j in range(3):
      @pl.when(j < n)
      def _(j=j):
        _idx_start(j, j)
    for j in range(2):
      @pl.when(j < n)
      def _(j=j):
        _idx_wait(j, j)
        _g_start(j)

    def _quad(m, _):
      k = R * m
      for j in range(R):
        _stage(k + j, j)
      return 0
    lax.fori_loop(0, (n + 2 + (R - 1)) // R, _quad, 0)

    pltpu.sync_copy(acc_sh.at[pl.ds(acc_base, NPX)], out_hbm.at[pl.ds(p0, NPX)])
    return 0
  lax.fori_loop(0, SB, _sub_block, 0)


@jax.jit
def kernel(indices, cnts, indptr, gamma, tokens):
  idx_p = jnp.concatenate([indices, jnp.zeros((3 * B,), jnp.int32)])
  cnt_p = jnp.concatenate([cnts, jnp.zeros((3 * B,), jnp.float32)])
  iptr_p = jnp.concatenate(
      [indptr, jnp.full((L - 1,), N_ENTRIES, jnp.int32)])

  mesh = plsc.VectorSubcoreMesh(
      core_axis_name="c", subcore_axis_name="s", num_cores=NC,
      num_subcores=NS)
  run = pl.kernel(
      _body,
      out_type=jax.ShapeDtypeStruct((N_PIXELS, D), jnp.float32),
      mesh=mesh,
      compiler_params=pltpu.CompilerParams(
          needs_layout_passes=False, use_tc_tiling_on_sc=False),
      scratch_types=[
          pltpu.VMEM((N_GENES,), jnp.float32),
          pltpu.VMEM((NPX,), jnp.int32),
          pltpu.VMEM((L,), jnp.int32),
          pltpu.VMEM((R, B), jnp.int32),
          pltpu.VMEM((R, B), jnp.float32),
          pltpu.VMEM((R, B), jnp.float32),
          pltpu.VMEM((R, B), jnp.int32),
          pltpu.VMEM((R, B, D), jnp.float32),
          pltpu.VMEM((ZR, D), jnp.float32),
          pltpu.VMEM_SHARED((NS * NPX, D), jnp.float32),
      ] + [pltpu.SemaphoreType.DMA] * 16,
  )
  out = run(idx_p, cnt_p, iptr_p, gamma, tokens)
  return out.reshape(H, W, D)
